# Initial kernel scaffold; baseline (speedup 1.0000x reference)
import jax
import jax.numpy as jnp
from jax.experimental import pallas as pl


def kernel(query, support, symbol_emb, W1, b1, W2, b2, ln_g, ln_b, W_ih, W_hh, b_ih, b_hh):
    raise NotImplementedError("write your pallas kernel here")


# speedup vs baseline: 1.5303x; 1.5303x over previous
import functools

import jax
import jax.numpy as jnp
from jax import lax
from jax.experimental import pallas as pl
from jax.experimental.pallas import tpu as pltpu
from jax.experimental.pallas import tpu_sc as plsc

D = 128
D2 = 2 * D
HID = 2 * D2
H4 = 4 * HID
B = 1024
FEW = 5
STEPS = 4


_NW = 32
_N_IDX = 2304
_PER_W = _N_IDX // _NW
_PAD_ROW = 100000


def _sc_gather_body(table_hbm, idx_hbm, out_hbm, idx_v, rows_v, sem):
    wid = lax.axis_index("s") * 2 + lax.axis_index("c")
    base = wid * _PER_W
    pltpu.sync_copy(idx_hbm.at[pl.ds(base, _PER_W)], idx_v)
    pltpu.async_copy(table_hbm.at[idx_v], rows_v, sem).wait()
    pltpu.sync_copy(rows_v, out_hbm.at[pl.ds(base, _PER_W)])


@functools.cache
def _make_sc_gather():
    return pl.kernel(
        _sc_gather_body,
        out_type=jax.ShapeDtypeStruct((_N_IDX, D), jnp.float32),
        mesh=plsc.VectorSubcoreMesh(core_axis_name="c", subcore_axis_name="s"),
        scratch_types=[
            pltpu.VMEM((_PER_W,), jnp.int32),
            pltpu.VMEM((_PER_W, D), jnp.float32),
            pltpu.SemaphoreType.DMA,
        ],
    )


def _sc_gather(table, idx):
    return _make_sc_gather()(table, idx)


_BT = 256
_NBT = B // _BT


def _encode(x, W1, b1, W2, b2, ln_g, ln_b):
    h = jnp.maximum(jnp.dot(x, W1, preferred_element_type=jnp.float32) + b1, 0.0)
    h = jnp.dot(h, W2, preferred_element_type=jnp.float32) + b2
    y = h + x
    mu = jnp.mean(y, axis=-1, keepdims=True)
    var = jnp.mean((y - mu) * (y - mu), axis=-1, keepdims=True)
    return ln_g * (y - mu) * lax.rsqrt(var + 1e-5) + ln_b


def _tc_body(q_ref, s_ref, W1_ref, b1_ref, W2_ref, b2_ref, lng_ref, lnb_ref,
             WihT_ref, WhhT_ref, bih_ref, bhh_ref, out_ref):
    W1 = W1_ref[...]
    b1 = b1_ref[...]
    W2 = W2_ref[...]
    b2 = b2_ref[...]
    ln_g = lng_ref[...]
    ln_b = lnb_ref[...]

    s_g = _encode(s_ref[...], W1, b1, W2, b2, ln_g, ln_b)
    row = lax.broadcasted_iota(jnp.int32, (8, 1), 0)
    s_g = jnp.where(row < FEW, s_g, 0.0)
    s_mean = jnp.sum(s_g, axis=0, keepdims=True) * (1.0 / FEW)

    q_g = _encode(q_ref[...], W1, b1, W2, b2, ln_g, ln_b)

    WihT = WihT_ref[...]
    WhhT_h = WhhT_ref[:D2, :]
    WhhT_r = WhhT_ref[D2:, :]

    a = (jnp.dot(q_g, WihT, preferred_element_type=jnp.float32)
         + bih_ref[...] + bhh_ref[...])
    r_row = jnp.dot(s_mean, WhhT_r, preferred_element_type=jnp.float32)

    c = jnp.zeros((_BT, HID), jnp.float32)
    h = None
    gates = a
    for step in range(STEPS):
        if step > 0:
            gates = a + r_row + jnp.dot(h, WhhT_h,
                                        preferred_element_type=jnp.float32)
        i = jax.nn.sigmoid(gates[:, :HID])
        f = jax.nn.sigmoid(gates[:, HID:2 * HID])
        g = jnp.tanh(gates[:, 2 * HID:3 * HID])
        o = jax.nn.sigmoid(gates[:, 3 * HID:])
        c = f * c + i * g if step > 0 else i * g
        h_lstm = o * jnp.tanh(c)
        h = q_g + h_lstm[:, :D2]

    out_ref[...] = jnp.sum(h * s_mean, axis=1, keepdims=True)


@jax.jit
def _tc_dense(q, s_pad, W1, b1, W2, b2, ln_g, ln_b, WihT, WhhT, b_ih, b_hh):
    full = lambda shape: pl.BlockSpec(shape, lambda i: (0,) * len(shape))
    return pl.pallas_call(
        _tc_body,
        grid=(_NBT,),
        in_specs=[
            pl.BlockSpec((_BT, D2), lambda i: (i, 0)),
            full((8, D2)),
            full((D2, 2 * D2)),
            full((1, 2 * D2)),
            full((2 * D2, D2)),
            full((1, D2)),
            full((1, D2)),
            full((1, D2)),
            full((D2, H4)),
            full((HID, H4)),
            full((1, H4)),
            full((1, H4)),
        ],
        out_specs=pl.BlockSpec((_BT, 1), lambda i: (i, 0)),
        out_shape=jax.ShapeDtypeStruct((B, 1), jnp.float32),
    )(q, s_pad, W1, b1, W2, b2, ln_g, ln_b, WihT, WhhT, b_ih, b_hh)


def kernel(query, support, symbol_emb, W1, b1, W2, b2, ln_g, ln_b, W_ih, W_hh, b_ih, b_hh):
    idx = jnp.concatenate([
        query.reshape(-1).astype(jnp.int32),
        support.reshape(-1).astype(jnp.int32),
        jnp.full((_N_IDX - 2 * B - 2 * FEW,), _PAD_ROW, jnp.int32),
    ])
    rows = _sc_gather(symbol_emb, idx)
    q = rows[:2 * B].reshape(B, D2)
    s = rows[2 * B:2 * B + 2 * FEW].reshape(FEW, D2)
    s_pad = jnp.zeros((8, D2), jnp.float32).at[:FEW].set(s)

    scores = _tc_dense(
        q, s_pad, W1, b1.reshape(1, -1), W2, b2.reshape(1, -1),
        ln_g.reshape(1, -1), ln_b.reshape(1, -1),
        W_ih.T, W_hh.T, b_ih.reshape(1, -1), b_hh.reshape(1, -1))
    return scores.reshape(B)


# speedup vs baseline: 1.5988x; 1.0448x over previous
import functools

import jax
import jax.numpy as jnp
from jax import lax
from jax.experimental import pallas as pl
from jax.experimental.pallas import tpu as pltpu
from jax.experimental.pallas import tpu_sc as plsc

D = 128
D2 = 2 * D
HID = 2 * D2
H4 = 4 * HID
B = 1024
FEW = 5
STEPS = 4


_NW = 32
_N_IDX = 2304
_PER_W = _N_IDX // _NW
_PAD_ROW = 100000


def _sc_gather_body(table_hbm, idx_hbm, out_hbm, idx_v, rows_v, sem):
    wid = lax.axis_index("s") * 2 + lax.axis_index("c")
    base = wid * _PER_W
    pltpu.sync_copy(idx_hbm.at[pl.ds(base, _PER_W)], idx_v)
    pltpu.async_copy(table_hbm.at[idx_v], rows_v, sem).wait()
    pltpu.sync_copy(rows_v, out_hbm.at[pl.ds(base, _PER_W)])


@functools.cache
def _make_sc_gather():
    return pl.kernel(
        _sc_gather_body,
        out_type=jax.ShapeDtypeStruct((_N_IDX, D), jnp.float32),
        mesh=plsc.VectorSubcoreMesh(core_axis_name="c", subcore_axis_name="s"),
        scratch_types=[
            pltpu.VMEM((_PER_W,), jnp.int32),
            pltpu.VMEM((_PER_W, D), jnp.float32),
            pltpu.SemaphoreType.DMA,
        ],
    )


def _sc_gather(table, idx):
    return _make_sc_gather()(table, idx)


_BT = 256
_NBT = B // _BT


def _encode(x, W1, b1, W2, b2, ln_g, ln_b):
    h = jnp.maximum(jnp.dot(x, W1, preferred_element_type=jnp.float32) + b1, 0.0)
    h = jnp.dot(h, W2, preferred_element_type=jnp.float32) + b2
    y = h + x
    mu = jnp.mean(y, axis=-1, keepdims=True)
    var = jnp.mean((y - mu) * (y - mu), axis=-1, keepdims=True)
    return ln_g * (y - mu) * lax.rsqrt(var + 1e-5) + ln_b


def _dot_nt(x, w):
    return lax.dot_general(x, w, (((1,), (1,)), ((), ())),
                           preferred_element_type=jnp.float32)


def _tc_body(q_ref, s_ref, W1_ref, b1_ref, W2_ref, b2_ref, lng_ref, lnb_ref,
             Wih_ref, Whh_ref, bih_ref, bhh_ref, out_ref):
    W1 = W1_ref[...]
    b1 = b1_ref[...]
    W2 = W2_ref[...]
    b2 = b2_ref[...]
    ln_g = lng_ref[...]
    ln_b = lnb_ref[...]

    s_g = _encode(s_ref[...], W1, b1, W2, b2, ln_g, ln_b)
    row = lax.broadcasted_iota(jnp.int32, (8, 1), 0)
    s_g = jnp.where(row < FEW, s_g, 0.0)
    s_mean = jnp.sum(s_g, axis=0, keepdims=True) * (1.0 / FEW)

    q_g = _encode(q_ref[...], W1, b1, W2, b2, ln_g, ln_b)

    Wih = Wih_ref[...]
    Whh_h = Whh_ref[:, :D2]
    Whh_r = Whh_ref[:, D2:]

    a = _dot_nt(q_g, Wih) + bih_ref[...] + bhh_ref[...]
    r_row = _dot_nt(s_mean, Whh_r)

    c = jnp.zeros((_BT, HID), jnp.float32)
    h = None
    gates = a
    for step in range(STEPS):
        if step > 0:
            gates = a + r_row + _dot_nt(h, Whh_h)
        i = jax.nn.sigmoid(gates[:, :HID])
        f = jax.nn.sigmoid(gates[:, HID:2 * HID])
        g = jnp.tanh(gates[:, 2 * HID:3 * HID])
        o = jax.nn.sigmoid(gates[:, 3 * HID:])
        c = f * c + i * g if step > 0 else i * g
        h_lstm = o * jnp.tanh(c)
        h = q_g + h_lstm[:, :D2]

    out_ref[...] = jnp.sum(h * s_mean, axis=1, keepdims=True)


@jax.jit
def _tc_dense(q, s_pad, W1, b1, W2, b2, ln_g, ln_b, W_ih, W_hh, b_ih, b_hh):
    full = lambda shape: pl.BlockSpec(shape, lambda i: (0,) * len(shape))
    return pl.pallas_call(
        _tc_body,
        grid=(_NBT,),
        in_specs=[
            pl.BlockSpec((_BT, D2), lambda i: (i, 0)),
            full((8, D2)),
            full((D2, 2 * D2)),
            full((1, 2 * D2)),
            full((2 * D2, D2)),
            full((1, D2)),
            full((1, D2)),
            full((1, D2)),
            full((H4, D2)),
            full((H4, HID)),
            full((1, H4)),
            full((1, H4)),
        ],
        out_specs=pl.BlockSpec((_BT, 1), lambda i: (i, 0)),
        out_shape=jax.ShapeDtypeStruct((B, 1), jnp.float32),
    )(q, s_pad, W1, b1, W2, b2, ln_g, ln_b, W_ih, W_hh, b_ih, b_hh)


def kernel(query, support, symbol_emb, W1, b1, W2, b2, ln_g, ln_b, W_ih, W_hh, b_ih, b_hh):
    idx = jnp.concatenate([
        query.reshape(-1).astype(jnp.int32),
        support.reshape(-1).astype(jnp.int32),
        jnp.full((_N_IDX - 2 * B - 2 * FEW,), _PAD_ROW, jnp.int32),
    ])
    rows = _sc_gather(symbol_emb, idx)
    q = rows[:2 * B].reshape(B, D2)
    s = rows[2 * B:2 * B + 2 * FEW].reshape(FEW, D2)
    s_pad = jnp.zeros((8, D2), jnp.float32).at[:FEW].set(s)

    scores = _tc_dense(
        q, s_pad, W1, b1.reshape(1, -1), W2, b2.reshape(1, -1),
        ln_g.reshape(1, -1), ln_b.reshape(1, -1),
        W_ih, W_hh, b_ih.reshape(1, -1), b_hh.reshape(1, -1))
    return scores.reshape(B)


# speedup vs baseline: 1.9294x; 1.2067x over previous
import functools

import jax
import jax.numpy as jnp
from jax import lax
from jax.experimental import pallas as pl
from jax.experimental.pallas import tpu as pltpu
from jax.experimental.pallas import tpu_sc as plsc

D = 128
D2 = 2 * D
HID = 2 * D2
H4 = 4 * HID
B = 1024
FEW = 5
STEPS = 4


_NW = 32
_NQ = 2 * B
_QPW = _NQ // _NW
_CH = 8
_NST = _QPW // _CH
_NS = 16


def _sc_gather_body(table_hbm, idxq_hbm, idxs_hbm, outq_hbm, outs_hbm,
                    idx_v, rows_v, idxs_v, rows_s, sem, sem_s):
    wid = lax.axis_index("s") * 2 + lax.axis_index("c")
    base = wid * _QPW
    pltpu.sync_copy(idxq_hbm.at[pl.ds(base, _QPW)], idx_v)
    copies = [
        pltpu.async_copy(
            table_hbm.at[idx_v.at[pl.ds(j * _CH, _CH)]],
            rows_v.at[pl.ds(j * _CH, _CH), :], sem)
        for j in range(_NST)
    ]

    @pl.when(wid == 0)
    def _():
        pltpu.sync_copy(idxs_hbm, idxs_v)
        pltpu.async_copy(table_hbm.at[idxs_v], rows_s, sem_s).wait()
        pltpu.sync_copy(rows_s, outs_hbm)

    for c in copies:
        c.wait()
    pltpu.sync_copy(rows_v, outq_hbm.at[pl.ds(base, _QPW)])


@functools.cache
def _make_sc_gather():
    return pl.kernel(
        _sc_gather_body,
        out_type=(
            jax.ShapeDtypeStruct((_NQ, D), jnp.float32),
            jax.ShapeDtypeStruct((_NS, D), jnp.float32),
        ),
        mesh=plsc.VectorSubcoreMesh(core_axis_name="c", subcore_axis_name="s"),
        scratch_types=[
            pltpu.VMEM((_QPW,), jnp.int32),
            pltpu.VMEM((_QPW, D), jnp.float32),
            pltpu.VMEM((_NS,), jnp.int32),
            pltpu.VMEM((_NS, D), jnp.float32),
            pltpu.SemaphoreType.DMA,
            pltpu.SemaphoreType.DMA,
        ],
    )


def _sc_gather(table, idx_q, idx_s):
    return _make_sc_gather()(table, idx_q, idx_s)


_BT = 256
_NBT = B // _BT


def _encode(x, W1, b1, W2, b2, ln_g, ln_b):
    h = jnp.maximum(jnp.dot(x, W1, preferred_element_type=jnp.float32) + b1, 0.0)
    h = jnp.dot(h, W2, preferred_element_type=jnp.float32) + b2
    y = h + x
    mu = jnp.mean(y, axis=-1, keepdims=True)
    var = jnp.mean((y - mu) * (y - mu), axis=-1, keepdims=True)
    return ln_g * (y - mu) * lax.rsqrt(var + 1e-5) + ln_b


def _dot_nt(x, w):
    return lax.dot_general(x, w, (((1,), (1,)), ((), ())),
                           preferred_element_type=jnp.float32)


def _tc_body(q_ref, s_ref, W1_ref, b1_ref, W2_ref, b2_ref, lng_ref, lnb_ref,
             Wih_ref, Whh_ref, bih_ref, bhh_ref, out_ref):
    W1 = W1_ref[...]
    b1 = b1_ref[...]
    W2 = W2_ref[...]
    b2 = b2_ref[...]
    ln_g = lng_ref[...]
    ln_b = lnb_ref[...]

    s_g = _encode(s_ref[...], W1, b1, W2, b2, ln_g, ln_b)
    row = lax.broadcasted_iota(jnp.int32, (8, 1), 0)
    s_g = jnp.where(row < FEW, s_g, 0.0)
    s_mean = jnp.sum(s_g, axis=0, keepdims=True) * (1.0 / FEW)

    q_g = _encode(q_ref[...], W1, b1, W2, b2, ln_g, ln_b)

    Wih = Wih_ref[...]
    Whh_h = Whh_ref[:, :D2]
    Whh_r = Whh_ref[:, D2:]

    a = _dot_nt(q_g, Wih) + bih_ref[...] + bhh_ref[...]
    r_row = _dot_nt(s_mean, Whh_r)

    c = jnp.zeros((_BT, HID), jnp.float32)
    h = None
    gates = a
    for step in range(STEPS):
        if step > 0:
            gates = a + r_row + _dot_nt(h, Whh_h)
        i = jax.nn.sigmoid(gates[:, :HID])
        f = jax.nn.sigmoid(gates[:, HID:2 * HID])
        g = jnp.tanh(gates[:, 2 * HID:3 * HID])
        o = jax.nn.sigmoid(gates[:, 3 * HID:])
        c = f * c + i * g if step > 0 else i * g
        h_lstm = o * jnp.tanh(c)
        h = q_g + h_lstm[:, :D2]

    out_ref[...] = jnp.sum(h * s_mean, axis=1, keepdims=True)


@jax.jit
def _tc_dense(q, s8, W1, b1, W2, b2, ln_g, ln_b, W_ih, W_hh, b_ih, b_hh):
    full = lambda shape: pl.BlockSpec(shape, lambda i: (0,) * len(shape))
    return pl.pallas_call(
        _tc_body,
        grid=(_NBT,),
        in_specs=[
            pl.BlockSpec((_BT, D2), lambda i: (i, 0)),
            full((8, D2)),
            full((D2, 2 * D2)),
            full((1, 2 * D2)),
            full((2 * D2, D2)),
            full((1, D2)),
            full((1, D2)),
            full((1, D2)),
            full((H4, D2)),
            full((H4, HID)),
            full((1, H4)),
            full((1, H4)),
        ],
        out_specs=pl.BlockSpec((_BT, 1), lambda i: (i, 0)),
        out_shape=jax.ShapeDtypeStruct((B, 1), jnp.float32),
    )(q, s8, W1, b1, W2, b2, ln_g, ln_b, W_ih, W_hh, b_ih, b_hh)


def kernel(query, support, symbol_emb, W1, b1, W2, b2, ln_g, ln_b, W_ih, W_hh, b_ih, b_hh):
    idx_q = query.reshape(-1).astype(jnp.int32)
    idx_s = jnp.concatenate([
        support.reshape(-1).astype(jnp.int32),
        jnp.zeros((_NS - 2 * FEW,), jnp.int32),
    ])
    rows_q, rows_s = _sc_gather(symbol_emb, idx_q, idx_s)
    q = rows_q.reshape(B, D2)
    s8 = rows_s.reshape(8, D2)

    scores = _tc_dense(
        q, s8, W1, b1.reshape(1, -1), W2, b2.reshape(1, -1),
        ln_g.reshape(1, -1), ln_b.reshape(1, -1),
        W_ih, W_hh, b_ih.reshape(1, -1), b_hh.reshape(1, -1))
    return scores.reshape(B)


# speedup vs baseline: 2.1844x; 1.1322x over previous
import functools

import jax
import jax.numpy as jnp
from jax import lax
from jax.experimental import pallas as pl
from jax.experimental.pallas import tpu as pltpu
from jax.experimental.pallas import tpu_sc as plsc

D = 128
D2 = 2 * D
HID = 2 * D2
H4 = 4 * HID
B = 1024
FEW = 5
STEPS = 4


_NW = 32
_NQ = 2 * B
_QPW = _NQ // _NW
_CH = 8
_NST = _QPW // _CH
_NS = 16


def _sc_gather_body(table_hbm, idxq_hbm, idxs_hbm, outq_hbm, outs_hbm,
                    idx_v, rows_v, idxs_v, rows_s, sem, sem_s):
    wid = lax.axis_index("s") * 2 + lax.axis_index("c")
    base = wid * _QPW
    pltpu.sync_copy(idxq_hbm.at[pl.ds(base, _QPW)], idx_v)
    copies = [
        pltpu.async_copy(
            table_hbm.at[idx_v.at[pl.ds(j * _CH, _CH)]],
            rows_v.at[pl.ds(j * _CH, _CH), :], sem)
        for j in range(_NST)
    ]

    @pl.when(wid == 0)
    def _():
        pltpu.sync_copy(idxs_hbm, idxs_v)
        pltpu.async_copy(table_hbm.at[idxs_v], rows_s, sem_s).wait()
        pltpu.sync_copy(rows_s, outs_hbm)

    for c in copies:
        c.wait()
    pltpu.sync_copy(rows_v, outq_hbm.at[pl.ds(base, _QPW)])


@functools.cache
def _make_sc_gather():
    return pl.kernel(
        _sc_gather_body,
        out_type=(
            jax.ShapeDtypeStruct((_NQ, D), jnp.float32),
            jax.ShapeDtypeStruct((_NS, D), jnp.float32),
        ),
        mesh=plsc.VectorSubcoreMesh(core_axis_name="c", subcore_axis_name="s"),
        scratch_types=[
            pltpu.VMEM((_QPW,), jnp.int32),
            pltpu.VMEM((_QPW, D), jnp.float32),
            pltpu.VMEM((_NS,), jnp.int32),
            pltpu.VMEM((_NS, D), jnp.float32),
            pltpu.SemaphoreType.DMA,
            pltpu.SemaphoreType.DMA,
        ],
    )


def _sc_gather(table, idx_q, idx_s):
    return _make_sc_gather()(table, idx_q, idx_s)


def _sigmoid(x):
    return 0.5 * jnp.tanh(0.5 * x) + 0.5


def _encode(x, W1, b1, W2, b2, ln_g, ln_b):
    h = jnp.maximum(jnp.dot(x, W1, preferred_element_type=jnp.float32) + b1, 0.0)
    h = jnp.dot(h, W2, preferred_element_type=jnp.float32) + b2
    y = h + x
    mu = jnp.mean(y, axis=-1, keepdims=True)
    var = jnp.mean((y - mu) * (y - mu), axis=-1, keepdims=True)
    return ln_g * (y - mu) * lax.rsqrt(var + 1e-5) + ln_b


def _dot_nt(x, w):
    return lax.dot_general(x, w, (((1,), (1,)), ((), ())),
                           preferred_element_type=jnp.float32)


def _tc_body(q_ref, s_ref, W1_ref, b1_ref, W2_ref, b2_ref, lng_ref, lnb_ref,
             Wih_hbm, Whh_hbm, bih_ref, bhh_ref, out_ref,
             wih_v, whh_v, sem_ih, sem_hh):
    cp_ih = pltpu.make_async_copy(Wih_hbm, wih_v, sem_ih)
    cp_hh = pltpu.make_async_copy(Whh_hbm, whh_v, sem_hh)
    cp_ih.start()
    cp_hh.start()

    W1 = W1_ref[...]
    b1 = b1_ref[...]
    W2 = W2_ref[...]
    b2 = b2_ref[...]
    ln_g = lng_ref[...]
    ln_b = lnb_ref[...]

    s_g = _encode(s_ref[...], W1, b1, W2, b2, ln_g, ln_b)
    row = lax.broadcasted_iota(jnp.int32, (8, 1), 0)
    s_g = jnp.where(row < FEW, s_g, 0.0)
    s_mean = jnp.sum(s_g, axis=0, keepdims=True) * (1.0 / FEW)

    q_g = _encode(q_ref[...], W1, b1, W2, b2, ln_g, ln_b)

    cp_ih.wait()
    a = _dot_nt(q_g, wih_v[...]) + bih_ref[...] + bhh_ref[...]

    cp_hh.wait()
    Whh_h = whh_v[:, :D2]
    Whh_r = whh_v[:, D2:]
    r_row = _dot_nt(s_mean, Whh_r)

    c = jnp.zeros((B, HID), jnp.float32)
    h = None
    gates = a
    for step in range(STEPS):
        if step > 0:
            gates = a + r_row + _dot_nt(h, Whh_h)
        o = _sigmoid(gates[:, 3 * HID:3 * HID + D2])
        if step < STEPS - 1:
            i = _sigmoid(gates[:, :HID])
            f = _sigmoid(gates[:, HID:2 * HID])
            g = jnp.tanh(gates[:, 2 * HID:3 * HID])
            c = f * c + i * g if step > 0 else i * g
        else:
            i = _sigmoid(gates[:, :D2])
            f = _sigmoid(gates[:, HID:HID + D2])
            g = jnp.tanh(gates[:, 2 * HID:2 * HID + D2])
            c = f * c[:, :D2] + i * g
        h = q_g + o * jnp.tanh(c[:, :D2])

    out_ref[...] = jnp.sum(h * s_mean, axis=1, keepdims=True)


@jax.jit
def _tc_dense(q, s8, W1, b1, W2, b2, ln_g, ln_b, W_ih, W_hh, b_ih, b_hh):
    full = lambda shape: pl.BlockSpec(shape, lambda *_: (0,) * len(shape))
    hbm = pl.BlockSpec(memory_space=pl.ANY)
    return pl.pallas_call(
        _tc_body,
        in_specs=[
            full((B, D2)),
            full((8, D2)),
            full((D2, 2 * D2)),
            full((1, 2 * D2)),
            full((2 * D2, D2)),
            full((1, D2)),
            full((1, D2)),
            full((1, D2)),
            hbm,
            hbm,
            full((1, H4)),
            full((1, H4)),
        ],
        out_specs=full((B, 1)),
        out_shape=jax.ShapeDtypeStruct((B, 1), jnp.float32),
        scratch_shapes=[
            pltpu.VMEM((H4, D2), jnp.float32),
            pltpu.VMEM((H4, HID), jnp.float32),
            pltpu.SemaphoreType.DMA,
            pltpu.SemaphoreType.DMA,
        ],
    )(q, s8, W1, b1, W2, b2, ln_g, ln_b, W_ih, W_hh, b_ih, b_hh)


def kernel(query, support, symbol_emb, W1, b1, W2, b2, ln_g, ln_b, W_ih, W_hh, b_ih, b_hh):
    idx_q = query.reshape(-1).astype(jnp.int32)
    idx_s = jnp.concatenate([
        support.reshape(-1).astype(jnp.int32),
        jnp.zeros((_NS - 2 * FEW,), jnp.int32),
    ])
    rows_q, rows_s = _sc_gather(symbol_emb, idx_q, idx_s)
    q = rows_q.reshape(B, D2)
    s8 = rows_s.reshape(8, D2)

    scores = _tc_dense(
        q, s8, W1, b1.reshape(1, -1), W2, b2.reshape(1, -1),
        ln_g.reshape(1, -1), ln_b.reshape(1, -1),
        W_ih, W_hh, b_ih.reshape(1, -1), b_hh.reshape(1, -1))
    return scores.reshape(B)
